# baseline (speedup 1.0000x reference)
import jax
import jax.numpy as jnp
from jax import lax
from jax.experimental import pallas as pl
from jax.experimental.pallas import tpu as pltpu
from jax.experimental.pallas import tpu_sc as plsc

N_NODES = 10000
N_EDGES = 160000
INDIM = 256
HIDDEN = 512
OUTDIM = 256
EDGEDIM = 16

NW = 32
GK2 = 80
IW = INDIM // 2
NBUF = 4
N_CC = 4
CW = HIDDEN // N_CC
N_TILES = 16
SK = 80
SNB = 3
RPT = 624
TAIL = N_NODES - N_TILES * RPT
ZR = 16
E_BLK = 1280

GROUPS = ((0, 40960), (40960, 39680), (80640, 39680), (120320, 39680))

_sc_mesh = plsc.VectorSubcoreMesh(core_axis_name="c", subcore_axis_name="s")


def _make_gather(ng):
    ncht = ng // GK2
    nslots = -(-ncht // NW)

    def body(x_hbm, row_hbm, col_hbm, sxx_hbm,
             ir0, ir1, ir2, ir3, ic0, ic1, ic2, ic3,
             bufr0, bufr1, bufr2, bufr3,
             bufc0, bufc1, bufc2, bufc3,
             isr0, isr1, isr2, isr3,
             isc0, isc1, isc2, isc3,
             gsr0, gsr1, gsr2, gsr3,
             gsc0, gsc1, gsc2, gsc3,
             wsr0, wsr1, wsr2, wsr3,
             wsc0, wsc1, wsc2, wsc3):
        cid = lax.axis_index("c")
        sid = lax.axis_index("s")
        wid = sid * 2 + cid
        idxr = (ir0, ir1, ir2, ir3)
        idxc = (ic0, ic1, ic2, ic3)
        bufr = (bufr0, bufr1, bufr2, bufr3)
        bufc = (bufc0, bufc1, bufc2, bufc3)
        isr = (isr0, isr1, isr2, isr3)
        isc = (isc0, isc1, isc2, isc3)
        gsr = (gsr0, gsr1, gsr2, gsr3)
        gsc = (gsc0, gsc1, gsc2, gsc3)
        wsr = (wsr0, wsr1, wsr2, wsr3)
        wsc = (wsc0, wsc1, wsc2, wsc3)

        def gc_of(t):
            return wid + t * NW

        def issue_idx(t, b):
            s = pl.ds(gc_of(t) * GK2, GK2)
            pltpu.async_copy(row_hbm.at[s], idxr[b], isr[b])
            pltpu.async_copy(col_hbm.at[s], idxc[b], isc[b])

        def wait_idx(b):
            pltpu.make_async_copy(row_hbm.at[pl.ds(0, GK2)], idxr[b], isr[b]).wait()
            pltpu.make_async_copy(col_hbm.at[pl.ds(0, GK2)], idxc[b], isc[b]).wait()

        def issue_gather(b):
            pltpu.async_copy(x_hbm.at[idxr[b]], bufr[b], gsr[b])
            pltpu.async_copy(x_hbm.at[idxc[b]], bufc[b], gsc[b])

        def wait_gather(b):
            pltpu.make_async_copy(x_hbm.at[idxr[b]], bufr[b], gsr[b]).wait()
            pltpu.make_async_copy(x_hbm.at[idxc[b]], bufc[b], gsc[b]).wait()

        def issue_writes(t, b):
            base = gc_of(t) * GK2
            pltpu.async_copy(bufr[b], sxx_hbm.at[pl.ds(base, GK2), pl.ds(0, IW)], wsr[b])
            pltpu.async_copy(bufc[b], sxx_hbm.at[pl.ds(base, GK2), pl.ds(IW, IW)], wsc[b])

        def drain_writes(b):
            pltpu.make_async_copy(bufr[b], sxx_hbm.at[pl.ds(0, GK2), pl.ds(0, IW)], wsr[b]).wait()
            pltpu.make_async_copy(bufc[b], sxx_hbm.at[pl.ds(0, GK2), pl.ds(IW, IW)], wsc[b]).wait()

        for b in range(NBUF):
            @pl.when(gc_of(b) < ncht)
            def _():
                issue_idx(b, b)

        def loop(g, carry):
            for b in range(NBUF):
                t = NBUF * g + b

                @pl.when(gc_of(t) < ncht)
                def _():
                    wait_idx(b)
                    issue_gather(b)
            for b in range(NBUF):
                t = NBUF * g + b

                @pl.when(gc_of(t) < ncht)
                def _():
                    wait_gather(b)
                    issue_writes(t, b)
            for b in range(NBUF):
                tn = NBUF * g + b + NBUF

                @pl.when(gc_of(tn) < ncht)
                def _():
                    drain_writes(b)
                    issue_idx(tn, b)
            return carry

        lax.fori_loop(0, -(-nslots // NBUF), loop, 0)
        for b in range(NBUF):
            @pl.when(gc_of(b) < ncht)
            def _():
                drain_writes(b)

    return pl.kernel(
        body,
        out_type=jax.ShapeDtypeStruct((ng, 2 * IW), jnp.int32),
        mesh=_sc_mesh,
        scratch_types=(
            [pltpu.VMEM((GK2,), jnp.int32)] * (2 * NBUF)
            + [pltpu.VMEM((GK2, IW), jnp.int32)] * (2 * NBUF)
            + [pltpu.SemaphoreType.DMA] * (6 * NBUF)
        ),
    )


_gathers = tuple(_make_gather(ng) for _, ng in GROUPS)


P_BLK = 1000


def _pack_body(x_ref, out_ref):
    lo = x_ref[:, :IW].astype(jnp.bfloat16).astype(jnp.float32)
    hi = x_ref[:, IW:].astype(jnp.bfloat16).astype(jnp.float32)
    lo_u = lax.bitcast_convert_type(lo, jnp.uint32)
    hi_u = lax.bitcast_convert_type(hi, jnp.uint32)
    word = (lo_u >> 16) | (hi_u & jnp.uint32(0xFFFF0000))
    out_ref[...] = lax.bitcast_convert_type(word, jnp.int32)


def _pack_x(x):
    return pl.pallas_call(
        _pack_body,
        grid=(N_NODES // P_BLK,),
        in_specs=[pl.BlockSpec((P_BLK, INDIM), lambda i: (i, 0))],
        out_specs=pl.BlockSpec((P_BLK, IW), lambda i: (i, 0)),
        out_shape=jax.ShapeDtypeStruct((N_NODES, IW), jnp.int32),
    )(x)


def _edge_body(sxx_ref, ea_ref, wlo_ref, whi_ref, w1c_ref, w2_ref,
               b1_ref, b2_ref, a1_ref, a2_ref, out_ref):
    w = lax.bitcast_convert_type(sxx_ref[...], jnp.uint32)
    lo = lax.bitcast_convert_type(w << 16, jnp.float32).astype(jnp.bfloat16)
    hi = lax.bitcast_convert_type(w & jnp.uint32(0xFFFF0000),
                                  jnp.float32).astype(jnp.bfloat16)
    z = jnp.dot(lo, wlo_ref[...], preferred_element_type=jnp.float32)
    z = z + jnp.dot(hi, whi_ref[...], preferred_element_type=jnp.float32)
    z = z + jnp.dot(ea_ref[...], w1c_ref[...], preferred_element_type=jnp.float32)
    z = z + b1_ref[...]
    a1 = a1_ref[0, 0]
    e1 = jnp.maximum(z, 0.0) + a1 * jnp.minimum(z, 0.0)
    z2 = jnp.dot(e1.astype(jnp.bfloat16), w2_ref[...],
                 preferred_element_type=jnp.float32) + b2_ref[...]
    a2 = a2_ref[0, 0]
    e2 = jnp.maximum(z2, 0.0) + a2 * jnp.minimum(z2, 0.0)
    for c in range(N_CC):
        out_ref[c] = e2[:, c * CW:(c + 1) * CW]


def _edge_mlp(sxx, ea, wlo, whi, w1c, w2, b1, b2, a1, a2):
    ng = sxx.shape[0]
    return pl.pallas_call(
        _edge_body,
        grid=(ng // E_BLK,),
        in_specs=[
            pl.BlockSpec((E_BLK, 2 * IW), lambda i: (i, 0)),
            pl.BlockSpec((E_BLK, EDGEDIM), lambda i: (i, 0)),
            pl.BlockSpec((INDIM, HIDDEN), lambda i: (0, 0)),
            pl.BlockSpec((INDIM, HIDDEN), lambda i: (0, 0)),
            pl.BlockSpec((EDGEDIM, HIDDEN), lambda i: (0, 0)),
            pl.BlockSpec((HIDDEN, HIDDEN), lambda i: (0, 0)),
            pl.BlockSpec((1, HIDDEN), lambda i: (0, 0)),
            pl.BlockSpec((1, HIDDEN), lambda i: (0, 0)),
            pl.BlockSpec((1, 1), lambda i: (0, 0)),
            pl.BlockSpec((1, 1), lambda i: (0, 0)),
        ],
        out_specs=pl.BlockSpec((N_CC, E_BLK, CW), lambda i: (0, i, 0)),
        out_shape=jax.ShapeDtypeStruct((N_CC, ng, CW), jnp.float32),
    )(sxx, ea, wlo, whi, w1c, w2, b1, b2, a1, a2)


def _make_scatter(ng):
    et = ng // N_TILES
    sch = et // SK

    def body(e2_hbm, row3d_hbm, agg_hbm,
             idx2d, ebuf0, ebuf1, zbuf, shared, es0, es1):
        cid = lax.axis_index("c")
        sid = lax.axis_index("s")
        ebuf = (ebuf0, ebuf1)
        es = (es0, es1)

        pltpu.sync_copy(row3d_hbm.at[sid], idx2d)

        def zloop(t, carry):
            i = t // 8
            j = t - i * 8
            zbuf[i, pl.ds(j * 16, 16)] = jnp.zeros((16,), jnp.float32)
            return carry

        lax.fori_loop(0, ZR * 8, zloop, 0)

        for cc in range(2):
            c_idx = cid * 2 + cc

            for k in range(39):
                pltpu.sync_copy(zbuf, shared.at[pl.ds(sid * RPT + k * ZR, ZR)])

            @pl.when(sid == N_TILES - 1)
            def _zero_tail():
                pltpu.sync_copy(zbuf.at[pl.ds(0, TAIL)],
                                shared.at[pl.ds(N_TILES * RPT, TAIL)])

            plsc.subcore_barrier()

            def issue_read(ch, b):
                pltpu.async_copy(e2_hbm.at[c_idx, pl.ds(sid * et + ch * SK, SK)],
                                 ebuf[b], es[b])

            def wait_read(b):
                pltpu.make_async_copy(e2_hbm.at[c_idx, pl.ds(sid * et, SK)],
                                      ebuf[b], es[b]).wait()

            issue_read(0, 0)
            issue_read(1, 1)

            def chunk(g, carry):
                for b in range(2):
                    ch = 2 * g + b

                    @pl.when(ch < sch)
                    def _():
                        wait_read(b)
                        pltpu.sync_copy(ebuf[b], shared.at[idx2d.at[ch]], add=True)

                        @pl.when(ch + 2 < sch)
                        def _():
                            issue_read(ch + 2, b)
                return carry

            lax.fori_loop(0, (sch + 1) // 2, chunk, 0)
            plsc.subcore_barrier()

            pltpu.sync_copy(shared.at[pl.ds(sid * RPT, RPT)],
                            agg_hbm.at[c_idx, pl.ds(sid * RPT, RPT)])

            @pl.when(sid == N_TILES - 1)
            def _write_tail():
                pltpu.sync_copy(shared.at[pl.ds(N_TILES * RPT, TAIL)],
                                agg_hbm.at[c_idx, pl.ds(N_TILES * RPT, TAIL)])

            plsc.subcore_barrier()

    return pl.kernel(
        body,
        out_type=jax.ShapeDtypeStruct((N_CC, N_NODES, CW), jnp.float32),
        mesh=_sc_mesh,
        scratch_types=(
            [pltpu.VMEM((sch, SK), jnp.int32)]
            + [pltpu.VMEM((SK, CW), jnp.float32)] * 2
            + [pltpu.VMEM((ZR, CW), jnp.float32),
               pltpu.VMEM_SHARED((N_NODES, CW), jnp.float32)]
            + [pltpu.SemaphoreType.DMA] * 2
        ),
    )


_scatters = tuple(_make_scatter(ng) for _, ng in GROUPS)


V_BLK = 1000


def _node_body(x_ref, *rest):
    aggr_refs = rest[:len(GROUPS)]
    (w3a_ref, w3b_ref, w4_ref, b3_ref, b4_ref, a3_ref, out_ref) = rest[len(GROUPS):]
    z = jnp.dot(x_ref[...], w3a_ref[...], preferred_element_type=jnp.float32)
    w3b = w3b_ref[...]
    for c in range(N_CC):
        agg_c = aggr_refs[0][c]
        for r in aggr_refs[1:]:
            agg_c = agg_c + r[c]
        z = z + jnp.dot(agg_c, w3b[c * CW:(c + 1) * CW, :],
                        preferred_element_type=jnp.float32)
    z = z + b3_ref[...]
    a3 = a3_ref[0, 0]
    h = jnp.maximum(z, 0.0) + a3 * jnp.minimum(z, 0.0)
    out_ref[...] = jnp.dot(h, w4_ref[...], preferred_element_type=jnp.float32) \
        + b4_ref[...]


def _node_mlp(x, aggrs, w3a, w3b, w4, b3, b4, a3):
    agg_spec = pl.BlockSpec((N_CC, V_BLK, CW), lambda i: (0, i, 0))
    return pl.pallas_call(
        _node_body,
        grid=(N_NODES // V_BLK,),
        in_specs=[
            pl.BlockSpec((V_BLK, INDIM), lambda i: (i, 0)),
        ] + [agg_spec] * len(GROUPS) + [
            pl.BlockSpec((INDIM, HIDDEN), lambda i: (0, 0)),
            pl.BlockSpec((HIDDEN, HIDDEN), lambda i: (0, 0)),
            pl.BlockSpec((HIDDEN, OUTDIM), lambda i: (0, 0)),
            pl.BlockSpec((1, HIDDEN), lambda i: (0, 0)),
            pl.BlockSpec((1, OUTDIM), lambda i: (0, 0)),
            pl.BlockSpec((1, 1), lambda i: (0, 0)),
        ],
        out_specs=pl.BlockSpec((V_BLK, OUTDIM), lambda i: (i, 0)),
        out_shape=jax.ShapeDtypeStruct((N_NODES, OUTDIM), jnp.float32),
    )(x, *aggrs, w3a, w3b, w4, b3, b4, a3)


def kernel(x, edge_index, edge_attr, W1, b1, a1, W2, b2, a2, W3, b3, a3, W4, b4):
    row = edge_index[0].astype(jnp.int32)
    col = edge_index[1].astype(jnp.int32)
    x32 = _pack_x(x)
    w1ab = W1[:, :2 * INDIM].T.astype(jnp.bfloat16)
    wlo = jnp.concatenate([w1ab[0:IW], w1ab[INDIM:INDIM + IW]], axis=0)
    whi = jnp.concatenate([w1ab[IW:INDIM], w1ab[INDIM + IW:]], axis=0)
    w1c = W1[:, 2 * INDIM:].T
    w2 = W2.T.astype(jnp.bfloat16)
    w3a = W3[:, :INDIM].T
    w3b = W3[:, INDIM:].T
    w4 = W4.T
    b1r = b1.reshape(1, HIDDEN)
    b2r = b2.reshape(1, HIDDEN)
    b3r = b3.reshape(1, HIDDEN)
    b4r = b4.reshape(1, OUTDIM)
    a1r = jnp.reshape(a1, (1, 1))
    a2r = jnp.reshape(a2, (1, 1))
    a3r = jnp.reshape(a3, (1, 1))

    aggs = []
    for gi, (start, ng) in enumerate(GROUPS):
        row_g = lax.dynamic_slice_in_dim(row, start, ng)
        col_g = lax.dynamic_slice_in_dim(col, start, ng)
        ea_g = lax.dynamic_slice_in_dim(edge_attr, start, ng)
        sxx32 = _gathers[gi](x32, row_g, col_g)
        e2 = _edge_mlp(sxx32, ea_g, wlo, whi, w1c, w2, b1r, b2r, a1r, a2r)
        row3d = row_g.reshape(N_TILES, ng // N_TILES // SK, SK)
        aggs.append(_scatters[gi](e2, row3d))

    return _node_mlp(x, aggs, w3a, w3b, w4, b3r, b4r, a3r)


# speedup vs baseline: 1.0569x; 1.0569x over previous
import jax
import jax.numpy as jnp
from jax import lax
from jax.experimental import pallas as pl
from jax.experimental.pallas import tpu as pltpu
from jax.experimental.pallas import tpu_sc as plsc

N_NODES = 10000
N_EDGES = 160000
INDIM = 256
HIDDEN = 512
OUTDIM = 256
EDGEDIM = 16

NW = 32
GK2 = 80
IW = INDIM // 2
NBUF = 4
N_CC = 4
CW = HIDDEN // N_CC
N_TILES = 16
SK = 80
SNB = 3
RPT = 624
TAIL = N_NODES - N_TILES * RPT
ZR = 16
E_BLK = 1280

GROUPS = ((0, 53760), (53760, 53760), (107520, 52480))

_sc_mesh = plsc.VectorSubcoreMesh(core_axis_name="c", subcore_axis_name="s")


def _make_gather(ng):
    ncht = ng // GK2
    nslots = -(-ncht // NW)

    def body(x_hbm, row_hbm, col_hbm, sxx_hbm,
             ir0, ir1, ir2, ir3, ic0, ic1, ic2, ic3,
             bufr0, bufr1, bufr2, bufr3,
             bufc0, bufc1, bufc2, bufc3,
             isr0, isr1, isr2, isr3,
             isc0, isc1, isc2, isc3,
             gsr0, gsr1, gsr2, gsr3,
             gsc0, gsc1, gsc2, gsc3,
             wsr0, wsr1, wsr2, wsr3,
             wsc0, wsc1, wsc2, wsc3):
        cid = lax.axis_index("c")
        sid = lax.axis_index("s")
        wid = sid * 2 + cid
        idxr = (ir0, ir1, ir2, ir3)
        idxc = (ic0, ic1, ic2, ic3)
        bufr = (bufr0, bufr1, bufr2, bufr3)
        bufc = (bufc0, bufc1, bufc2, bufc3)
        isr = (isr0, isr1, isr2, isr3)
        isc = (isc0, isc1, isc2, isc3)
        gsr = (gsr0, gsr1, gsr2, gsr3)
        gsc = (gsc0, gsc1, gsc2, gsc3)
        wsr = (wsr0, wsr1, wsr2, wsr3)
        wsc = (wsc0, wsc1, wsc2, wsc3)

        def gc_of(t):
            return wid + t * NW

        def issue_idx(t, b):
            s = pl.ds(gc_of(t) * GK2, GK2)
            pltpu.async_copy(row_hbm.at[s], idxr[b], isr[b])
            pltpu.async_copy(col_hbm.at[s], idxc[b], isc[b])

        def wait_idx(b):
            pltpu.make_async_copy(row_hbm.at[pl.ds(0, GK2)], idxr[b], isr[b]).wait()
            pltpu.make_async_copy(col_hbm.at[pl.ds(0, GK2)], idxc[b], isc[b]).wait()

        def issue_gather(b):
            pltpu.async_copy(x_hbm.at[idxr[b]], bufr[b], gsr[b])
            pltpu.async_copy(x_hbm.at[idxc[b]], bufc[b], gsc[b])

        def wait_gather(b):
            pltpu.make_async_copy(x_hbm.at[idxr[b]], bufr[b], gsr[b]).wait()
            pltpu.make_async_copy(x_hbm.at[idxc[b]], bufc[b], gsc[b]).wait()

        def issue_writes(t, b):
            base = gc_of(t) * GK2
            pltpu.async_copy(bufr[b], sxx_hbm.at[pl.ds(base, GK2), pl.ds(0, IW)], wsr[b])
            pltpu.async_copy(bufc[b], sxx_hbm.at[pl.ds(base, GK2), pl.ds(IW, IW)], wsc[b])

        def drain_writes(b):
            pltpu.make_async_copy(bufr[b], sxx_hbm.at[pl.ds(0, GK2), pl.ds(0, IW)], wsr[b]).wait()
            pltpu.make_async_copy(bufc[b], sxx_hbm.at[pl.ds(0, GK2), pl.ds(IW, IW)], wsc[b]).wait()

        for b in range(NBUF):
            @pl.when(gc_of(b) < ncht)
            def _():
                issue_idx(b, b)

        def loop(g, carry):
            for b in range(NBUF):
                t = NBUF * g + b

                @pl.when(gc_of(t) < ncht)
                def _():
                    wait_idx(b)
                    issue_gather(b)
            for b in range(NBUF):
                t = NBUF * g + b

                @pl.when(gc_of(t) < ncht)
                def _():
                    wait_gather(b)
                    issue_writes(t, b)
            for b in range(NBUF):
                tn = NBUF * g + b + NBUF

                @pl.when(gc_of(tn) < ncht)
                def _():
                    drain_writes(b)
                    issue_idx(tn, b)
            return carry

        lax.fori_loop(0, -(-nslots // NBUF), loop, 0)
        for b in range(NBUF):
            @pl.when(gc_of(b) < ncht)
            def _():
                drain_writes(b)

    return pl.kernel(
        body,
        out_type=jax.ShapeDtypeStruct((ng, 2 * IW), jnp.int32),
        mesh=_sc_mesh,
        scratch_types=(
            [pltpu.VMEM((GK2,), jnp.int32)] * (2 * NBUF)
            + [pltpu.VMEM((GK2, IW), jnp.int32)] * (2 * NBUF)
            + [pltpu.SemaphoreType.DMA] * (6 * NBUF)
        ),
    )


_gathers = tuple(_make_gather(ng) for _, ng in GROUPS)


P_BLK = 1000


def _pack_body(x_ref, out_ref):
    lo = x_ref[:, :IW].astype(jnp.bfloat16).astype(jnp.float32)
    hi = x_ref[:, IW:].astype(jnp.bfloat16).astype(jnp.float32)
    lo_u = lax.bitcast_convert_type(lo, jnp.uint32)
    hi_u = lax.bitcast_convert_type(hi, jnp.uint32)
    word = (lo_u >> 16) | (hi_u & jnp.uint32(0xFFFF0000))
    out_ref[...] = lax.bitcast_convert_type(word, jnp.int32)


def _pack_x(x):
    return pl.pallas_call(
        _pack_body,
        grid=(N_NODES // P_BLK,),
        in_specs=[pl.BlockSpec((P_BLK, INDIM), lambda i: (i, 0))],
        out_specs=pl.BlockSpec((P_BLK, IW), lambda i: (i, 0)),
        out_shape=jax.ShapeDtypeStruct((N_NODES, IW), jnp.int32),
    )(x)


def _edge_body(sxx_ref, ea_ref, wlo_ref, whi_ref, w1c_ref, w2_ref,
               b1_ref, b2_ref, a1_ref, a2_ref, out_ref):
    w = lax.bitcast_convert_type(sxx_ref[...], jnp.uint32)
    lo = lax.bitcast_convert_type(w << 16, jnp.float32).astype(jnp.bfloat16)
    hi = lax.bitcast_convert_type(w & jnp.uint32(0xFFFF0000),
                                  jnp.float32).astype(jnp.bfloat16)
    z = jnp.dot(lo, wlo_ref[...], preferred_element_type=jnp.float32)
    z = z + jnp.dot(hi, whi_ref[...], preferred_element_type=jnp.float32)
    z = z + jnp.dot(ea_ref[...], w1c_ref[...], preferred_element_type=jnp.float32)
    z = z + b1_ref[...]
    a1 = a1_ref[0, 0]
    e1 = jnp.maximum(z, 0.0) + a1 * jnp.minimum(z, 0.0)
    z2 = jnp.dot(e1.astype(jnp.bfloat16), w2_ref[...],
                 preferred_element_type=jnp.float32) + b2_ref[...]
    a2 = a2_ref[0, 0]
    e2 = jnp.maximum(z2, 0.0) + a2 * jnp.minimum(z2, 0.0)
    for c in range(N_CC):
        out_ref[c] = e2[:, c * CW:(c + 1) * CW]


def _edge_mlp(sxx, ea, wlo, whi, w1c, w2, b1, b2, a1, a2):
    ng = sxx.shape[0]
    return pl.pallas_call(
        _edge_body,
        grid=(ng // E_BLK,),
        in_specs=[
            pl.BlockSpec((E_BLK, 2 * IW), lambda i: (i, 0)),
            pl.BlockSpec((E_BLK, EDGEDIM), lambda i: (i, 0)),
            pl.BlockSpec((INDIM, HIDDEN), lambda i: (0, 0)),
            pl.BlockSpec((INDIM, HIDDEN), lambda i: (0, 0)),
            pl.BlockSpec((EDGEDIM, HIDDEN), lambda i: (0, 0)),
            pl.BlockSpec((HIDDEN, HIDDEN), lambda i: (0, 0)),
            pl.BlockSpec((1, HIDDEN), lambda i: (0, 0)),
            pl.BlockSpec((1, HIDDEN), lambda i: (0, 0)),
            pl.BlockSpec((1, 1), lambda i: (0, 0)),
            pl.BlockSpec((1, 1), lambda i: (0, 0)),
        ],
        out_specs=pl.BlockSpec((N_CC, E_BLK, CW), lambda i: (0, i, 0)),
        out_shape=jax.ShapeDtypeStruct((N_CC, ng, CW), jnp.float32),
    )(sxx, ea, wlo, whi, w1c, w2, b1, b2, a1, a2)


def _make_scatter(ng):
    et = ng // N_TILES
    sch = et // SK

    def body(e2_hbm, row3d_hbm, agg_hbm,
             idx2d, ebuf0, ebuf1, zbuf, shared, es0, es1):
        cid = lax.axis_index("c")
        sid = lax.axis_index("s")
        ebuf = (ebuf0, ebuf1)
        es = (es0, es1)

        pltpu.sync_copy(row3d_hbm.at[sid], idx2d)

        def zloop(t, carry):
            i = t // 8
            j = t - i * 8
            zbuf[i, pl.ds(j * 16, 16)] = jnp.zeros((16,), jnp.float32)
            return carry

        lax.fori_loop(0, ZR * 8, zloop, 0)

        for cc in range(2):
            c_idx = cid * 2 + cc

            for k in range(39):
                pltpu.sync_copy(zbuf, shared.at[pl.ds(sid * RPT + k * ZR, ZR)])

            @pl.when(sid == N_TILES - 1)
            def _zero_tail():
                pltpu.sync_copy(zbuf.at[pl.ds(0, TAIL)],
                                shared.at[pl.ds(N_TILES * RPT, TAIL)])

            plsc.subcore_barrier()

            def issue_read(ch, b):
                pltpu.async_copy(e2_hbm.at[c_idx, pl.ds(sid * et + ch * SK, SK)],
                                 ebuf[b], es[b])

            def wait_read(b):
                pltpu.make_async_copy(e2_hbm.at[c_idx, pl.ds(sid * et, SK)],
                                      ebuf[b], es[b]).wait()

            issue_read(0, 0)
            issue_read(1, 1)

            def chunk(g, carry):
                for b in range(2):
                    ch = 2 * g + b

                    @pl.when(ch < sch)
                    def _():
                        wait_read(b)
                        pltpu.sync_copy(ebuf[b], shared.at[idx2d.at[ch]], add=True)

                        @pl.when(ch + 2 < sch)
                        def _():
                            issue_read(ch + 2, b)
                return carry

            lax.fori_loop(0, (sch + 1) // 2, chunk, 0)
            plsc.subcore_barrier()

            pltpu.sync_copy(shared.at[pl.ds(sid * RPT, RPT)],
                            agg_hbm.at[c_idx, pl.ds(sid * RPT, RPT)])

            @pl.when(sid == N_TILES - 1)
            def _write_tail():
                pltpu.sync_copy(shared.at[pl.ds(N_TILES * RPT, TAIL)],
                                agg_hbm.at[c_idx, pl.ds(N_TILES * RPT, TAIL)])

            plsc.subcore_barrier()

    return pl.kernel(
        body,
        out_type=jax.ShapeDtypeStruct((N_CC, N_NODES, CW), jnp.float32),
        mesh=_sc_mesh,
        scratch_types=(
            [pltpu.VMEM((sch, SK), jnp.int32)]
            + [pltpu.VMEM((SK, CW), jnp.float32)] * 2
            + [pltpu.VMEM((ZR, CW), jnp.float32),
               pltpu.VMEM_SHARED((N_NODES, CW), jnp.float32)]
            + [pltpu.SemaphoreType.DMA] * 2
        ),
    )


_scatters = tuple(_make_scatter(ng) for _, ng in GROUPS)


V_BLK = 1000


def _node_body(x_ref, *rest):
    aggr_refs = rest[:len(GROUPS)]
    (w3a_ref, w3b_ref, w4_ref, b3_ref, b4_ref, a3_ref, out_ref) = rest[len(GROUPS):]
    z = jnp.dot(x_ref[...], w3a_ref[...], preferred_element_type=jnp.float32)
    w3b = w3b_ref[...]
    for c in range(N_CC):
        agg_c = aggr_refs[0][c]
        for r in aggr_refs[1:]:
            agg_c = agg_c + r[c]
        z = z + jnp.dot(agg_c, w3b[c * CW:(c + 1) * CW, :],
                        preferred_element_type=jnp.float32)
    z = z + b3_ref[...]
    a3 = a3_ref[0, 0]
    h = jnp.maximum(z, 0.0) + a3 * jnp.minimum(z, 0.0)
    out_ref[...] = jnp.dot(h, w4_ref[...], preferred_element_type=jnp.float32) \
        + b4_ref[...]


def _node_mlp(x, aggrs, w3a, w3b, w4, b3, b4, a3):
    agg_spec = pl.BlockSpec((N_CC, V_BLK, CW), lambda i: (0, i, 0))
    return pl.pallas_call(
        _node_body,
        grid=(N_NODES // V_BLK,),
        in_specs=[
            pl.BlockSpec((V_BLK, INDIM), lambda i: (i, 0)),
        ] + [agg_spec] * len(GROUPS) + [
            pl.BlockSpec((INDIM, HIDDEN), lambda i: (0, 0)),
            pl.BlockSpec((HIDDEN, HIDDEN), lambda i: (0, 0)),
            pl.BlockSpec((HIDDEN, OUTDIM), lambda i: (0, 0)),
            pl.BlockSpec((1, HIDDEN), lambda i: (0, 0)),
            pl.BlockSpec((1, OUTDIM), lambda i: (0, 0)),
            pl.BlockSpec((1, 1), lambda i: (0, 0)),
        ],
        out_specs=pl.BlockSpec((V_BLK, OUTDIM), lambda i: (i, 0)),
        out_shape=jax.ShapeDtypeStruct((N_NODES, OUTDIM), jnp.float32),
    )(x, *aggrs, w3a, w3b, w4, b3, b4, a3)


def kernel(x, edge_index, edge_attr, W1, b1, a1, W2, b2, a2, W3, b3, a3, W4, b4):
    row = edge_index[0].astype(jnp.int32)
    col = edge_index[1].astype(jnp.int32)
    x32 = _pack_x(x)
    w1ab = W1[:, :2 * INDIM].T.astype(jnp.bfloat16)
    wlo = jnp.concatenate([w1ab[0:IW], w1ab[INDIM:INDIM + IW]], axis=0)
    whi = jnp.concatenate([w1ab[IW:INDIM], w1ab[INDIM + IW:]], axis=0)
    w1c = W1[:, 2 * INDIM:].T
    w2 = W2.T.astype(jnp.bfloat16)
    w3a = W3[:, :INDIM].T
    w3b = W3[:, INDIM:].T
    w4 = W4.T
    b1r = b1.reshape(1, HIDDEN)
    b2r = b2.reshape(1, HIDDEN)
    b3r = b3.reshape(1, HIDDEN)
    b4r = b4.reshape(1, OUTDIM)
    a1r = jnp.reshape(a1, (1, 1))
    a2r = jnp.reshape(a2, (1, 1))
    a3r = jnp.reshape(a3, (1, 1))

    aggs = []
    for gi, (start, ng) in enumerate(GROUPS):
        row_g = lax.dynamic_slice_in_dim(row, start, ng)
        col_g = lax.dynamic_slice_in_dim(col, start, ng)
        ea_g = lax.dynamic_slice_in_dim(edge_attr, start, ng)
        sxx32 = _gathers[gi](x32, row_g, col_g)
        e2 = _edge_mlp(sxx32, ea_g, wlo, whi, w1c, w2, b1r, b2r, a1r, a2r)
        row3d = row_g.reshape(N_TILES, ng // N_TILES // SK, SK)
        aggs.append(_scatters[gi](e2, row3d))

    return _node_mlp(x, aggs, w3a, w3b, w4, b3r, b4r, a3r)
